# Initial kernel scaffold; baseline (speedup 1.0000x reference)
import jax
import jax.numpy as jnp
from jax.experimental import pallas as pl


def kernel(input, embedding, pos_embedding):
    raise NotImplementedError("write your pallas kernel here")


# speedup vs baseline: 1.6642x; 1.6642x over previous
import functools

import jax
import jax.numpy as jnp
from jax import lax
from jax.experimental import pallas as pl
from jax.experimental.pallas import tpu as pltpu
from jax.experimental.pallas import tpu_sc as plsc

P = 50
D = 512
B = 4096
N = B * P
NW = 32
ROWS_PER_W = N // NW
CH = 64
NCH = ROWS_PER_W // CH


def _build_table_tc(inp_ref, emb_ref, pos_ref, t_ref, idx_ref):
    emb = emb_ref[...]
    pos = pos_ref[...]
    t_ref[...] = pos[:, None, :] + emb[None, :, :]
    idx_ref[...] = inp_ref[...] + P * lax.broadcasted_iota(jnp.int32, (B, P), 1)


def _sc_gather(t_hbm, idx_hbm, out_hbm, idx_v, buf, gsem):
    wid = lax.axis_index("s") * 2 + lax.axis_index("c")
    base = wid * ROWS_PER_W
    pltpu.sync_copy(idx_hbm.at[pl.ds(base, ROWS_PER_W)], idx_v)

    def body(c, carry):
        off = pl.multiple_of(c * CH, CH)
        pltpu.async_copy(t_hbm.at[idx_v.at[pl.ds(off, CH)]], buf, gsem).wait()
        pltpu.sync_copy(buf, out_hbm.at[pl.ds(base + off, CH)])
        return carry

    lax.fori_loop(0, NCH, body, 0)


def kernel(input, embedding, pos_embedding):
    t, idx = pl.pallas_call(
        _build_table_tc,
        out_shape=(
            jax.ShapeDtypeStruct((P, P, D), jnp.float32),
            jax.ShapeDtypeStruct((B, P), jnp.int32),
        ),
    )(input.astype(jnp.int32), embedding, pos_embedding)

    t = t.reshape(P * P, D)
    idx_flat = idx.reshape(N)

    sc = functools.partial(
        pl.kernel,
        out_type=jax.ShapeDtypeStruct((N, D), jnp.float32),
        mesh=plsc.VectorSubcoreMesh(
            core_axis_name="c", subcore_axis_name="s",
            num_cores=2, num_subcores=16),
        scratch_types=[
            pltpu.VMEM((ROWS_PER_W,), jnp.int32),
            pltpu.VMEM((CH, D), jnp.float32),
            pltpu.SemaphoreType.DMA,
        ],
    )(_sc_gather)

    out_flat = sc(t, idx_flat)
    return out_flat.reshape(B, P, D)


# speedup vs baseline: 1.7672x; 1.0619x over previous
import functools

import jax
import jax.numpy as jnp
from jax import lax
from jax.experimental import pallas as pl
from jax.experimental.pallas import tpu as pltpu
from jax.experimental.pallas import tpu_sc as plsc

P = 50
D = 512
B = 4096
N = B * P
NW = 32
ROWS_PER_W = N // NW
CH = 64
NCH = ROWS_PER_W // CH


def _build_table_tc(inp_ref, emb_ref, pos_ref, t_ref, idx_ref):
    emb = emb_ref[...]
    pos = pos_ref[...]
    t_ref[...] = pos[:, None, :] + emb[None, :, :]
    idx_ref[...] = inp_ref[...] + P * lax.broadcasted_iota(jnp.int32, (B, P), 1)


def _sc_gather(t_hbm, idx_hbm, out_hbm, idx_v, buf0, buf1, g0, g1, s0, s1):
    wid = lax.axis_index("s") * 2 + lax.axis_index("c")
    base = wid * ROWS_PER_W
    pltpu.sync_copy(idx_hbm.at[pl.ds(base, ROWS_PER_W)], idx_v)

    bufs = (buf0, buf1)
    gsems = (g0, g1)
    ssems = (s0, s1)

    def gather(c, slot):
        off = pl.multiple_of(c * CH, CH)
        return pltpu.make_async_copy(
            t_hbm.at[idx_v.at[pl.ds(off, CH)]], bufs[slot], gsems[slot])

    def store(c, slot):
        off = pl.multiple_of(c * CH, CH)
        return pltpu.make_async_copy(
            bufs[slot], out_hbm.at[pl.ds(base + off, CH)], ssems[slot])

    gather(0, 0).start()

    def body(g, carry):
        c0 = 2 * g

        @pl.when(g > 0)
        def _():
            store(c0 - 1, 1).wait()

        gather(c0 + 1, 1).start()
        gather(c0, 0).wait()
        store(c0, 0).start()

        @pl.when(g < NCH // 2 - 1)
        def _():
            store(c0, 0).wait()
            gather(c0 + 2, 0).start()

        gather(c0 + 1, 1).wait()
        store(c0 + 1, 1).start()
        return carry

    lax.fori_loop(0, NCH // 2, body, 0)
    store(NCH - 2, 0).wait()
    store(NCH - 1, 1).wait()


def kernel(input, embedding, pos_embedding):
    t, idx = pl.pallas_call(
        _build_table_tc,
        out_shape=(
            jax.ShapeDtypeStruct((P, P, D), jnp.float32),
            jax.ShapeDtypeStruct((B, P), jnp.int32),
        ),
    )(input.astype(jnp.int32), embedding, pos_embedding)

    t = t.reshape(P * P, D)
    idx_flat = idx.reshape(N)

    sc = functools.partial(
        pl.kernel,
        out_type=jax.ShapeDtypeStruct((N, D), jnp.float32),
        mesh=plsc.VectorSubcoreMesh(
            core_axis_name="c", subcore_axis_name="s",
            num_cores=2, num_subcores=16),
        scratch_types=[
            pltpu.VMEM((ROWS_PER_W,), jnp.int32),
            pltpu.VMEM((CH, D), jnp.float32),
            pltpu.VMEM((CH, D), jnp.float32),
            pltpu.SemaphoreType.DMA,
            pltpu.SemaphoreType.DMA,
            pltpu.SemaphoreType.DMA,
            pltpu.SemaphoreType.DMA,
        ],
    )(_sc_gather)

    out_flat = sc(t, idx_flat)
    return out_flat.reshape(B, P, D)
